# Initial kernel scaffold; baseline (speedup 1.0000x reference)
import jax
import jax.numpy as jnp
from jax.experimental import pallas as pl


def kernel(x, true_vals, W_ih, W_hh, b_ih, b_hh, W1, b1, W2, b2, W3, b3, starts, ends):
    raise NotImplementedError("write your pallas kernel here")


# speedup vs baseline: 32.6732x; 32.6732x over previous
import jax
import jax.numpy as jnp
from jax.experimental import pallas as pl
from jax.experimental.pallas import tpu as pltpu

_B, _T, _D, _H = 64, 256, 256, 512


def _lstm_kernel(starts_ref, ends_ref, x_ref, wih_ref, bg_ref, whh_ref,
                 w1_ref, b1_ref, w2_ref, b2_ref, w3_ref, b3_ref,
                 out_ref, xp_ref, hc_ref, hall_ref):
    b = pl.program_id(0)

    xp_ref[...] = (jnp.dot(x_ref[0], wih_ref[...],
                           preferred_element_type=jnp.float32) + bg_ref[...])

    @pl.when(b == 0)
    def _():
        hc_ref[...] = jnp.zeros_like(hc_ref)

    start = starts_ref[b]
    end = ends_ref[b]

    def step(t, carry):
        h, c = carry
        gates = xp_ref[pl.ds(t, 1), :] + jnp.dot(
            h, whh_ref[...], preferred_element_type=jnp.float32)
        i = gates[:, 0:_H]
        f = gates[:, _H:2 * _H]
        g = gates[:, 2 * _H:3 * _H]
        o = gates[:, 3 * _H:4 * _H]
        c2 = jax.nn.sigmoid(f) * c + jax.nn.sigmoid(i) * jnp.tanh(g)
        h2 = jax.nn.sigmoid(o) * jnp.tanh(c2)
        return (h2, c2)

    h, c = jax.lax.fori_loop(start + 1, end + 1, step,
                             (hc_ref[0:1, :], hc_ref[1:2, :]))
    hc_ref[0:1, :] = h
    hc_ref[1:2, :] = c
    hall_ref[pl.ds(b, 1), :, :] = h.reshape(1, 1, _H)

    @pl.when(b == _B - 1)
    def _():
        hall = hall_ref[...].reshape(_B, _H)
        z = jnp.maximum(
            jnp.dot(hall, w1_ref[...], preferred_element_type=jnp.float32)
            + b1_ref[...], 0.0)
        z = jnp.maximum(
            jnp.dot(z, w2_ref[...], preferred_element_type=jnp.float32)
            + b2_ref[...], 0.0)
        logit = jnp.sum(z * w3_ref[...], axis=1, keepdims=True) + b3_ref[...]
        out_ref[...] = jax.nn.sigmoid(logit)


def kernel(x, true_vals, W_ih, W_hh, b_ih, b_hh, W1, b1, W2, b2, W3, b3,
           starts, ends):
    B, T, D = x.shape
    H = W_hh.shape[1]
    bg = (b_ih + b_hh).reshape(1, 4 * H)

    grid_spec = pltpu.PrefetchScalarGridSpec(
        num_scalar_prefetch=2,
        grid=(B,),
        in_specs=[
            pl.BlockSpec((1, T, D), lambda b, s, e: (b, 0, 0)),
            pl.BlockSpec((D, 4 * H), lambda b, s, e: (0, 0)),
            pl.BlockSpec((1, 4 * H), lambda b, s, e: (0, 0)),
            pl.BlockSpec((H, 4 * H), lambda b, s, e: (0, 0)),
            pl.BlockSpec((H, 2 * H), lambda b, s, e: (0, 0)),
            pl.BlockSpec((1, 2 * H), lambda b, s, e: (0, 0)),
            pl.BlockSpec((2 * H, 2 * H), lambda b, s, e: (0, 0)),
            pl.BlockSpec((1, 2 * H), lambda b, s, e: (0, 0)),
            pl.BlockSpec((1, 2 * H), lambda b, s, e: (0, 0)),
            pl.BlockSpec((1, 1), lambda b, s, e: (0, 0)),
        ],
        out_specs=pl.BlockSpec((B, 1), lambda b, s, e: (0, 0)),
        scratch_shapes=[
            pltpu.VMEM((T, 4 * H), jnp.float32),
            pltpu.VMEM((2, H), jnp.float32),
            pltpu.VMEM((B, 1, H), jnp.float32),
        ],
    )
    predict = pl.pallas_call(
        _lstm_kernel,
        grid_spec=grid_spec,
        out_shape=jax.ShapeDtypeStruct((B, 1), jnp.float32),
        compiler_params=pltpu.CompilerParams(
            dimension_semantics=("arbitrary",),
            vmem_limit_bytes=48 * 1024 * 1024,
        ),
        name="lstm_tagger",
    )(starts.astype(jnp.int32), ends.astype(jnp.int32), x, W_ih.T, bg,
      W_hh.T, W1.T, b1.reshape(1, -1), W2.T, b2.reshape(1, -1),
      W3.reshape(1, -1), b3.reshape(1, 1))
    return predict, true_vals


# speedup vs baseline: 32.7782x; 1.0032x over previous
import jax
import jax.numpy as jnp
from jax.experimental import pallas as pl
from jax.experimental.pallas import tpu as pltpu

_B, _T, _D, _H = 64, 256, 256, 512


def _lstm_kernel(starts_ref, ends_ref, x_ref, wih_ref, bg_ref, whh_ref,
                 w1_ref, b1_ref, w2_ref, b2_ref, w3_ref, b3_ref,
                 out_ref, xp_ref, hc_ref, hall_ref):
    b = pl.program_id(0)

    xp_ref[...] = (jnp.dot(x_ref[0], wih_ref[...],
                           preferred_element_type=jnp.float32) + bg_ref[...])

    @pl.when(b == 0)
    def _():
        hc_ref[...] = jnp.zeros_like(hc_ref)

    start = starts_ref[b]
    end = ends_ref[b]

    def step(t, carry):
        h, c = carry
        gates = xp_ref[pl.ds(t, 1), :] + jnp.dot(
            h.astype(jnp.bfloat16), whh_ref[...],
            preferred_element_type=jnp.float32)
        i = gates[:, 0:_H]
        f = gates[:, _H:2 * _H]
        g = gates[:, 2 * _H:3 * _H]
        o = gates[:, 3 * _H:4 * _H]
        c2 = jax.nn.sigmoid(f) * c + jax.nn.sigmoid(i) * jnp.tanh(g)
        h2 = jax.nn.sigmoid(o) * jnp.tanh(c2)
        return (h2, c2)

    h, c = jax.lax.fori_loop(start + 1, end + 1, step,
                             (hc_ref[0:1, :], hc_ref[1:2, :]))
    hc_ref[0:1, :] = h
    hc_ref[1:2, :] = c
    hall_ref[pl.ds(b, 1), :, :] = h.reshape(1, 1, _H)

    @pl.when(b == _B - 1)
    def _():
        hall = hall_ref[...].reshape(_B, _H)
        z = jnp.maximum(
            jnp.dot(hall.astype(jnp.bfloat16), w1_ref[...],
                    preferred_element_type=jnp.float32)
            + b1_ref[...], 0.0)
        z = jnp.maximum(
            jnp.dot(z.astype(jnp.bfloat16), w2_ref[...],
                    preferred_element_type=jnp.float32)
            + b2_ref[...], 0.0)
        logit = jnp.sum(z * w3_ref[...], axis=1, keepdims=True) + b3_ref[...]
        out_ref[...] = jax.nn.sigmoid(logit)


def kernel(x, true_vals, W_ih, W_hh, b_ih, b_hh, W1, b1, W2, b2, W3, b3,
           starts, ends):
    B, T, D = x.shape
    H = W_hh.shape[1]
    bg = (b_ih + b_hh).reshape(1, 4 * H)

    grid_spec = pltpu.PrefetchScalarGridSpec(
        num_scalar_prefetch=2,
        grid=(B,),
        in_specs=[
            pl.BlockSpec((1, T, D), lambda b, s, e: (b, 0, 0)),
            pl.BlockSpec((D, 4 * H), lambda b, s, e: (0, 0)),
            pl.BlockSpec((1, 4 * H), lambda b, s, e: (0, 0)),
            pl.BlockSpec((H, 4 * H), lambda b, s, e: (0, 0)),
            pl.BlockSpec((H, 2 * H), lambda b, s, e: (0, 0)),
            pl.BlockSpec((1, 2 * H), lambda b, s, e: (0, 0)),
            pl.BlockSpec((2 * H, 2 * H), lambda b, s, e: (0, 0)),
            pl.BlockSpec((1, 2 * H), lambda b, s, e: (0, 0)),
            pl.BlockSpec((1, 2 * H), lambda b, s, e: (0, 0)),
            pl.BlockSpec((1, 1), lambda b, s, e: (0, 0)),
        ],
        out_specs=pl.BlockSpec((B, 1), lambda b, s, e: (0, 0)),
        scratch_shapes=[
            pltpu.VMEM((T, 4 * H), jnp.float32),
            pltpu.VMEM((2, H), jnp.float32),
            pltpu.VMEM((B, 1, H), jnp.float32),
        ],
    )
    predict = pl.pallas_call(
        _lstm_kernel,
        grid_spec=grid_spec,
        out_shape=jax.ShapeDtypeStruct((B, 1), jnp.float32),
        compiler_params=pltpu.CompilerParams(
            dimension_semantics=("arbitrary",),
            vmem_limit_bytes=48 * 1024 * 1024,
        ),
        name="lstm_tagger",
    )(starts.astype(jnp.int32), ends.astype(jnp.int32),
      x.astype(jnp.bfloat16), W_ih.T.astype(jnp.bfloat16), bg,
      W_hh.T.astype(jnp.bfloat16), W1.T.astype(jnp.bfloat16),
      b1.reshape(1, -1), W2.T.astype(jnp.bfloat16), b2.reshape(1, -1),
      W3.reshape(1, -1), b3.reshape(1, 1))
    return predict, true_vals


# speedup vs baseline: 34.3380x; 1.0476x over previous
import jax
import jax.numpy as jnp
from jax.experimental import pallas as pl
from jax.experimental.pallas import tpu as pltpu

_B, _T, _D, _H = 64, 256, 256, 512


def _lstm_kernel(starts_ref, ends_ref, x_ref, wih_ref, bg_ref, whh_ref,
                 w1_ref, b1_ref, w2_ref, b2_ref, w3_ref, b3_ref,
                 out_ref, xp_ref, hc_ref, hall_ref):
    b = pl.program_id(0)

    xp_ref[...] = (jnp.dot(x_ref[0], wih_ref[...],
                           preferred_element_type=jnp.float32) + bg_ref[...])

    @pl.when(b == 0)
    def _():
        hc_ref[...] = jnp.zeros_like(hc_ref)

    start = starts_ref[b]
    end = ends_ref[b]

    def one_step(t, h, c):
        gates = xp_ref[pl.ds(t, 1), :] + jnp.dot(
            h.astype(jnp.bfloat16), whh_ref[...],
            preferred_element_type=jnp.float32)
        i = gates[:, 0:_H]
        f = gates[:, _H:2 * _H]
        g = gates[:, 2 * _H:3 * _H]
        o = gates[:, 3 * _H:4 * _H]
        c2 = jax.nn.sigmoid(f) * c + jax.nn.sigmoid(i) * jnp.tanh(g)
        h2 = jax.nn.sigmoid(o) * jnp.tanh(c2)
        return h2, c2

    n_active = end - start

    def step2(idx, carry):
        h, c = carry
        t = start + 1 + 2 * idx
        h, c = one_step(t, h, c)
        h, c = one_step(t + 1, h, c)
        return (h, c)

    h, c = jax.lax.fori_loop(0, n_active // 2, step2,
                             (hc_ref[0:1, :], hc_ref[1:2, :]))
    h, c = jax.lax.cond(n_active % 2 == 1,
                        lambda hc: one_step(end, hc[0], hc[1]),
                        lambda hc: hc, (h, c))
    hc_ref[0:1, :] = h
    hc_ref[1:2, :] = c
    hall_ref[pl.ds(b, 1), :, :] = h.reshape(1, 1, _H)

    @pl.when(b == _B - 1)
    def _():
        hall = hall_ref[...].reshape(_B, _H)
        z = jnp.maximum(
            jnp.dot(hall.astype(jnp.bfloat16), w1_ref[...],
                    preferred_element_type=jnp.float32)
            + b1_ref[...], 0.0)
        z = jnp.maximum(
            jnp.dot(z.astype(jnp.bfloat16), w2_ref[...],
                    preferred_element_type=jnp.float32)
            + b2_ref[...], 0.0)
        logit = jnp.sum(z * w3_ref[...], axis=1, keepdims=True) + b3_ref[...]
        out_ref[...] = jax.nn.sigmoid(logit)


def kernel(x, true_vals, W_ih, W_hh, b_ih, b_hh, W1, b1, W2, b2, W3, b3,
           starts, ends):
    B, T, D = x.shape
    H = W_hh.shape[1]
    bg = (b_ih + b_hh).reshape(1, 4 * H)

    grid_spec = pltpu.PrefetchScalarGridSpec(
        num_scalar_prefetch=2,
        grid=(B,),
        in_specs=[
            pl.BlockSpec((1, T, D), lambda b, s, e: (b, 0, 0)),
            pl.BlockSpec((D, 4 * H), lambda b, s, e: (0, 0)),
            pl.BlockSpec((1, 4 * H), lambda b, s, e: (0, 0)),
            pl.BlockSpec((H, 4 * H), lambda b, s, e: (0, 0)),
            pl.BlockSpec((H, 2 * H), lambda b, s, e: (0, 0)),
            pl.BlockSpec((1, 2 * H), lambda b, s, e: (0, 0)),
            pl.BlockSpec((2 * H, 2 * H), lambda b, s, e: (0, 0)),
            pl.BlockSpec((1, 2 * H), lambda b, s, e: (0, 0)),
            pl.BlockSpec((1, 2 * H), lambda b, s, e: (0, 0)),
            pl.BlockSpec((1, 1), lambda b, s, e: (0, 0)),
        ],
        out_specs=pl.BlockSpec((B, 1), lambda b, s, e: (0, 0)),
        scratch_shapes=[
            pltpu.VMEM((T, 4 * H), jnp.float32),
            pltpu.VMEM((2, H), jnp.float32),
            pltpu.VMEM((B, 1, H), jnp.float32),
        ],
    )
    predict = pl.pallas_call(
        _lstm_kernel,
        grid_spec=grid_spec,
        out_shape=jax.ShapeDtypeStruct((B, 1), jnp.float32),
        compiler_params=pltpu.CompilerParams(
            dimension_semantics=("arbitrary",),
            vmem_limit_bytes=48 * 1024 * 1024,
        ),
        name="lstm_tagger",
    )(starts.astype(jnp.int32), ends.astype(jnp.int32),
      x.astype(jnp.bfloat16), W_ih.T.astype(jnp.bfloat16), bg,
      W_hh.T.astype(jnp.bfloat16), W1.T.astype(jnp.bfloat16),
      b1.reshape(1, -1), W2.T.astype(jnp.bfloat16), b2.reshape(1, -1),
      W3.reshape(1, -1), b3.reshape(1, 1))
    return predict, true_vals


# speedup vs baseline: 37.9552x; 1.1053x over previous
import jax
import jax.numpy as jnp
from jax.experimental import pallas as pl
from jax.experimental.pallas import tpu as pltpu

_B, _T, _D, _H = 64, 256, 256, 512


def _proj_kernel(x_ref, w_ref, bg_ref, xp_ref):
    xp_ref[...] = (jnp.dot(x_ref[...], w_ref[...],
                           preferred_element_type=jnp.float32)
                   + bg_ref[...])


def _project(x2d, wih_t, bg):
    rows = _B * _T
    blk = 2048
    return pl.pallas_call(
        _proj_kernel,
        grid=(rows // blk,),
        in_specs=[
            pl.BlockSpec((blk, _D), lambda i: (i, 0)),
            pl.BlockSpec((_D, 4 * _H), lambda i: (0, 0)),
            pl.BlockSpec((1, 4 * _H), lambda i: (0, 0)),
        ],
        out_specs=pl.BlockSpec((blk, 4 * _H), lambda i: (i, 0)),
        out_shape=jax.ShapeDtypeStruct((rows, 4 * _H), jnp.float32),
        compiler_params=pltpu.CompilerParams(
            dimension_semantics=("arbitrary",),
            vmem_limit_bytes=48 * 1024 * 1024,
        ),
        name="lstm_proj",
    )(x2d, wih_t, bg)


def _rec_kernel(starts_ref, ends_ref, xp_ref, wt_ref, hall_ref, hc_ref):
    b = pl.program_id(0)

    @pl.when(b == 0)
    def _():
        hc_ref[...] = jnp.zeros_like(hc_ref)

    for m in (0, 1):
        for addr in (0, 4, 8, 12, 16):
            pltpu.matmul_pop(addr, (16, 256), jnp.float32, mxu_index=m)

    pltpu.matmul_push_rhs(wt_ref[0], staging_register=0, mxu_index=0)
    pltpu.matmul_push_rhs(wt_ref[8], staging_register=0, mxu_index=1)
    pltpu.matmul_push_rhs(wt_ref[1], staging_register=1, mxu_index=0)
    pltpu.matmul_push_rhs(wt_ref[9], staging_register=1, mxu_index=1)

    start = starts_ref[b]
    end = ends_ref[b]

    def step(t, carry):
        h, c = carry
        hb = h.astype(jnp.bfloat16)
        h16 = (jnp.broadcast_to(hb[:, 0:256], (16, 256)),
               jnp.broadcast_to(hb[:, 256:512], (16, 256)))
        for i in range(8):
            n, k = divmod(i, 2)
            j = (i + 2) % 8
            for m in (0, 1):
                pltpu.matmul_acc_lhs(acc_addr=4 * n, lhs=h16[k],
                                     mxu_index=m, load_staged_rhs=i % 2)
                pltpu.matmul_push_rhs(wt_ref[m * 8 + j],
                                      staging_register=i % 2, mxu_index=m)
        chunk = {}
        for n in range(4):
            for m in (0, 1):
                chunk[(m, n)] = pltpu.matmul_pop(
                    4 * n, (16, 256), jnp.float32, mxu_index=m)[0:1, :]
        xr = xp_ref[0, pl.ds(t, 1), :]
        gi = jnp.concatenate([chunk[(0, 0)], chunk[(0, 1)]], axis=1) + xr[:, 0:512]
        gf = jnp.concatenate([chunk[(0, 2)], chunk[(0, 3)]], axis=1) + xr[:, 512:1024]
        gg = jnp.concatenate([chunk[(1, 0)], chunk[(1, 1)]], axis=1) + xr[:, 1024:1536]
        go = jnp.concatenate([chunk[(1, 2)], chunk[(1, 3)]], axis=1) + xr[:, 1536:2048]
        c2 = jax.nn.sigmoid(gf) * c + jax.nn.sigmoid(gi) * jnp.tanh(gg)
        h2 = jax.nn.sigmoid(go) * jnp.tanh(c2)
        return (h2, c2)

    h, c = jax.lax.fori_loop(start + 1, end + 1, step,
                             (hc_ref[0:1, :], hc_ref[1:2, :]))
    hc_ref[0:1, :] = h
    hc_ref[1:2, :] = c
    hall_ref[...] = h.reshape(1, 1, _H)

    dummy = jnp.zeros((16, 256), jnp.bfloat16)
    for m in (0, 1):
        pltpu.matmul_acc_lhs(acc_addr=16, lhs=dummy, mxu_index=m,
                             load_staged_rhs=0)
        pltpu.matmul_acc_lhs(acc_addr=16, lhs=dummy, mxu_index=m,
                             load_staged_rhs=1)
        pltpu.matmul_pop(16, (16, 256), jnp.float32, mxu_index=m)


def _recur(starts, ends, xp3, wt):
    grid_spec = pltpu.PrefetchScalarGridSpec(
        num_scalar_prefetch=2,
        grid=(_B,),
        in_specs=[
            pl.BlockSpec((1, _T, 4 * _H), lambda b, s, e: (b, 0, 0)),
            pl.BlockSpec((16, 256, 256), lambda b, s, e: (0, 0, 0)),
        ],
        out_specs=pl.BlockSpec((1, 1, _H), lambda b, s, e: (b, 0, 0)),
        scratch_shapes=[pltpu.VMEM((2, _H), jnp.float32)],
    )
    return pl.pallas_call(
        _rec_kernel,
        grid_spec=grid_spec,
        out_shape=jax.ShapeDtypeStruct((_B, 1, _H), jnp.float32),
        compiler_params=pltpu.CompilerParams(
            dimension_semantics=("arbitrary",),
            vmem_limit_bytes=48 * 1024 * 1024,
        ),
        name="lstm_recur",
    )(starts, ends, xp3, wt)


def _mlp_kernel(h_ref, w1_ref, b1_ref, w2_ref, b2_ref, w3_ref, b3_ref,
                out_ref):
    z = jnp.maximum(
        jnp.dot(h_ref[...].astype(jnp.bfloat16), w1_ref[...],
                preferred_element_type=jnp.float32) + b1_ref[...], 0.0)
    z = jnp.maximum(
        jnp.dot(z.astype(jnp.bfloat16), w2_ref[...],
                preferred_element_type=jnp.float32) + b2_ref[...], 0.0)
    logit = jnp.sum(z * w3_ref[...], axis=1, keepdims=True) + b3_ref[...]
    out_ref[...] = jax.nn.sigmoid(logit)


def _mlp(hall, w1t, b1, w2t, b2, w3, b3):
    return pl.pallas_call(
        _mlp_kernel,
        out_shape=jax.ShapeDtypeStruct((_B, 1), jnp.float32),
        compiler_params=pltpu.CompilerParams(
            vmem_limit_bytes=48 * 1024 * 1024,
        ),
        name="lstm_mlp",
    )(hall, w1t, b1, w2t, b2, w3, b3)


def kernel(x, true_vals, W_ih, W_hh, b_ih, b_hh, W1, b1, W2, b2, W3, b3,
           starts, ends):
    bg = (b_ih + b_hh).reshape(1, 4 * _H)
    x2d = x.reshape(_B * _T, _D).astype(jnp.bfloat16)
    xp = _project(x2d, W_ih.T.astype(jnp.bfloat16), bg)
    xp3 = xp.reshape(_B, _T, 4 * _H)
    wt = (W_hh.T.astype(jnp.bfloat16)
          .reshape(2, 256, 8, 256).transpose(2, 0, 1, 3)
          .reshape(16, 256, 256))
    hall = _recur(starts.astype(jnp.int32), ends.astype(jnp.int32), xp3, wt)
    predict = _mlp(hall.reshape(_B, _H), W1.T.astype(jnp.bfloat16),
                   b1.reshape(1, -1), W2.T.astype(jnp.bfloat16),
                   b2.reshape(1, -1), W3.reshape(1, -1), b3.reshape(1, 1))
    return predict, true_vals


# speedup vs baseline: 39.2189x; 1.0333x over previous
import jax
import jax.numpy as jnp
from jax.experimental import pallas as pl
from jax.experimental.pallas import tpu as pltpu

_B, _T, _D, _H = 64, 256, 256, 512

_TILE_ORDER = ((0, 0), (0, 1), (1, 0), (1, 1), (2, 0), (2, 1), (6, 0), (6, 1),
               (3, 0), (3, 1), (4, 0), (4, 1), (5, 0), (5, 1), (7, 0), (7, 1))


def _sig(x):
    return jnp.tanh(x * 0.5) * 0.5 + 0.5


def _proj_kernel(x_ref, w_ref, bg_ref, xp_ref):
    xp_ref[...] = (jnp.dot(x_ref[...], w_ref[...],
                           preferred_element_type=jnp.float32)
                   + bg_ref[...])


def _project(x2d, wih_t, bg):
    rows = _B * _T
    blk = 2048
    return pl.pallas_call(
        _proj_kernel,
        grid=(rows // blk,),
        in_specs=[
            pl.BlockSpec((blk, _D), lambda i: (i, 0)),
            pl.BlockSpec((_D, 4 * _H), lambda i: (0, 0)),
            pl.BlockSpec((1, 4 * _H), lambda i: (0, 0)),
        ],
        out_specs=pl.BlockSpec((blk, 4 * _H), lambda i: (i, 0)),
        out_shape=jax.ShapeDtypeStruct((rows, 4 * _H), jnp.float32),
        compiler_params=pltpu.CompilerParams(
            dimension_semantics=("arbitrary",),
            vmem_limit_bytes=48 * 1024 * 1024,
        ),
        name="lstm_proj",
    )(x2d, wih_t, bg)


def _rec_kernel(starts_ref, ends_ref, xp_ref, wt_ref, hall_ref, hc_ref):
    b = pl.program_id(0)

    @pl.when(b == 0)
    def _():
        hc_ref[...] = jnp.zeros_like(hc_ref)

    for m in (0, 1):
        for addr in (0, 4, 8, 12, 16):
            pltpu.matmul_pop(addr, (16, 256), jnp.float32, mxu_index=m)

    pltpu.matmul_push_rhs(wt_ref[0], staging_register=0, mxu_index=0)
    pltpu.matmul_push_rhs(wt_ref[8], staging_register=0, mxu_index=1)
    pltpu.matmul_push_rhs(wt_ref[1], staging_register=1, mxu_index=0)
    pltpu.matmul_push_rhs(wt_ref[9], staging_register=1, mxu_index=1)

    start = starts_ref[b]
    end = ends_ref[b]

    def step(t, carry):
        h, c = carry
        hb = h.astype(jnp.bfloat16)
        h16 = (jnp.broadcast_to(hb[:, 0:256], (16, 256)),
               jnp.broadcast_to(hb[:, 256:512], (16, 256)))
        for i in range(8):
            n, k = divmod(i, 2)
            j = (i + 2) % 8
            for m in (0, 1):
                pltpu.matmul_acc_lhs(acc_addr=4 * n, lhs=h16[k],
                                     mxu_index=m, load_staged_rhs=i % 2)
                pltpu.matmul_push_rhs(wt_ref[m * 8 + j],
                                      staging_register=i % 2, mxu_index=m)
        chunk = {}
        for n in range(4):
            for m in (0, 1):
                chunk[(m, n)] = pltpu.matmul_pop(
                    4 * n, (16, 256), jnp.float32, mxu_index=m)[0:1, :]
        xr = xp_ref[0, pl.ds(t, 1), :]
        gi = jnp.concatenate([chunk[(0, 0)], chunk[(0, 1)]], axis=1) + xr[:, 0:512]
        gf = jnp.concatenate([chunk[(0, 2)], chunk[(1, 0)]], axis=1) + xr[:, 512:1024]
        gg = jnp.concatenate([chunk[(1, 1)], chunk[(1, 2)]], axis=1) + xr[:, 1024:1536]
        go = jnp.concatenate([chunk[(0, 3)], chunk[(1, 3)]], axis=1) + xr[:, 1536:2048]
        c2 = _sig(gf) * c + _sig(gi) * jnp.tanh(gg)
        h2 = _sig(go) * jnp.tanh(c2)
        return (h2, c2)

    h, c = jax.lax.fori_loop(start + 1, end + 1, step,
                             (hc_ref[0:1, :], hc_ref[1:2, :]))
    hc_ref[0:1, :] = h
    hc_ref[1:2, :] = c
    hall_ref[...] = h.reshape(1, 1, _H)

    dummy = jnp.zeros((16, 256), jnp.bfloat16)
    for m in (0, 1):
        pltpu.matmul_acc_lhs(acc_addr=16, lhs=dummy, mxu_index=m,
                             load_staged_rhs=0)
        pltpu.matmul_acc_lhs(acc_addr=16, lhs=dummy, mxu_index=m,
                             load_staged_rhs=1)
        pltpu.matmul_pop(16, (16, 256), jnp.float32, mxu_index=m)


def _recur(starts, ends, xp3, wt):
    grid_spec = pltpu.PrefetchScalarGridSpec(
        num_scalar_prefetch=2,
        grid=(_B,),
        in_specs=[
            pl.BlockSpec((1, _T, 4 * _H), lambda b, s, e: (b, 0, 0)),
            pl.BlockSpec((16, 256, 256), lambda b, s, e: (0, 0, 0)),
        ],
        out_specs=pl.BlockSpec((1, 1, _H), lambda b, s, e: (b, 0, 0)),
        scratch_shapes=[pltpu.VMEM((2, _H), jnp.float32)],
    )
    return pl.pallas_call(
        _rec_kernel,
        grid_spec=grid_spec,
        out_shape=jax.ShapeDtypeStruct((_B, 1, _H), jnp.float32),
        compiler_params=pltpu.CompilerParams(
            dimension_semantics=("arbitrary",),
            vmem_limit_bytes=48 * 1024 * 1024,
        ),
        name="lstm_recur",
    )(starts, ends, xp3, wt)


def _mlp_kernel(h_ref, w1_ref, b1_ref, w2_ref, b2_ref, w3_ref, b3_ref,
                out_ref):
    z = jnp.maximum(
        jnp.dot(h_ref[...].astype(jnp.bfloat16), w1_ref[...],
                preferred_element_type=jnp.float32) + b1_ref[...], 0.0)
    z = jnp.maximum(
        jnp.dot(z.astype(jnp.bfloat16), w2_ref[...],
                preferred_element_type=jnp.float32) + b2_ref[...], 0.0)
    logit = jnp.sum(z * w3_ref[...], axis=1, keepdims=True) + b3_ref[...]
    out_ref[...] = jax.nn.sigmoid(logit)


def _mlp(hall, w1t, b1, w2t, b2, w3, b3):
    return pl.pallas_call(
        _mlp_kernel,
        out_shape=jax.ShapeDtypeStruct((_B, 1), jnp.float32),
        compiler_params=pltpu.CompilerParams(
            vmem_limit_bytes=48 * 1024 * 1024,
        ),
        name="lstm_mlp",
    )(hall, w1t, b1, w2t, b2, w3, b3)


def kernel(x, true_vals, W_ih, W_hh, b_ih, b_hh, W1, b1, W2, b2, W3, b3,
           starts, ends):
    bg = (b_ih + b_hh).reshape(1, 4 * _H)
    x2d = x.reshape(_B * _T, _D).astype(jnp.bfloat16)
    xp = _project(x2d, W_ih.T.astype(jnp.bfloat16), bg)
    xp3 = xp.reshape(_B, _T, 4 * _H)
    whh_t = W_hh.T.astype(jnp.bfloat16)
    wt = jnp.stack([whh_t[256 * k:256 * (k + 1), 256 * j:256 * (j + 1)]
                    for j, k in _TILE_ORDER])
    hall = _recur(starts.astype(jnp.int32), ends.astype(jnp.int32), xp3, wt)
    predict = _mlp(hall.reshape(_B, _H), W1.T.astype(jnp.bfloat16),
                   b1.reshape(1, -1), W2.T.astype(jnp.bfloat16),
                   b2.reshape(1, -1), W3.reshape(1, -1), b3.reshape(1, 1))
    return predict, true_vals


# speedup vs baseline: 39.2297x; 1.0003x over previous
import jax
import jax.numpy as jnp
from jax.experimental import pallas as pl
from jax.experimental.pallas import tpu as pltpu

_B, _T, _D, _H = 64, 256, 256, 512

_TILE_ORDER = ((0, 0), (0, 1), (1, 0), (1, 1), (2, 0), (2, 1), (6, 0), (6, 1),
               (3, 0), (3, 1), (4, 0), (4, 1), (5, 0), (5, 1), (7, 0), (7, 1))


def _sig(x):
    return jnp.tanh(x * 0.5) * 0.5 + 0.5


def _proj_kernel(x_ref, w_ref, bg_ref, xp_ref):
    xp_ref[...] = (jnp.dot(x_ref[...], w_ref[...],
                           preferred_element_type=jnp.float32)
                   + bg_ref[...])


def _project(x2d, wih_t, bg):
    rows = _B * _T
    blk = 2048
    return pl.pallas_call(
        _proj_kernel,
        grid=(rows // blk,),
        in_specs=[
            pl.BlockSpec((blk, _D), lambda i: (i, 0)),
            pl.BlockSpec((_D, 4 * _H), lambda i: (0, 0)),
            pl.BlockSpec((1, 4 * _H), lambda i: (0, 0)),
        ],
        out_specs=pl.BlockSpec((blk, 4 * _H), lambda i: (i, 0)),
        out_shape=jax.ShapeDtypeStruct((rows, 4 * _H), jnp.float32),
        compiler_params=pltpu.CompilerParams(
            dimension_semantics=("arbitrary",),
            vmem_limit_bytes=48 * 1024 * 1024,
        ),
        name="lstm_proj",
    )(x2d, wih_t, bg)


def _rec_kernel(starts_ref, ends_ref, xp_ref, wt_ref, hall_ref, hc_ref):
    b = pl.program_id(0)

    @pl.when(b == 0)
    def _():
        hc_ref[...] = jnp.zeros_like(hc_ref)

    for m in (0, 1):
        for addr in (0, 4, 8, 12, 16):
            pltpu.matmul_pop(addr, (16, 256), jnp.float32, mxu_index=m)

    pltpu.matmul_push_rhs(wt_ref[0], staging_register=0, mxu_index=0)
    pltpu.matmul_push_rhs(wt_ref[8], staging_register=0, mxu_index=1)
    pltpu.matmul_push_rhs(wt_ref[1], staging_register=1, mxu_index=0)
    pltpu.matmul_push_rhs(wt_ref[9], staging_register=1, mxu_index=1)

    start = starts_ref[b]
    end = ends_ref[b]

    def one_step(t, h, c):
        hb = h.astype(jnp.bfloat16)
        h16 = (jnp.broadcast_to(hb[:, 0:256], (16, 256)),
               jnp.broadcast_to(hb[:, 256:512], (16, 256)))
        for i in range(8):
            n, k = divmod(i, 2)
            j = (i + 2) % 8
            for m in (0, 1):
                pltpu.matmul_acc_lhs(acc_addr=4 * n, lhs=h16[k],
                                     mxu_index=m, load_staged_rhs=i % 2)
                pltpu.matmul_push_rhs(wt_ref[m * 8 + j],
                                      staging_register=i % 2, mxu_index=m)
        chunk = {}
        for n in range(4):
            for m in (0, 1):
                chunk[(m, n)] = pltpu.matmul_pop(
                    4 * n, (16, 256), jnp.float32, mxu_index=m)[0:1, :]
        xr = xp_ref[0, pl.ds(t, 1), :]
        gi = jnp.concatenate([chunk[(0, 0)], chunk[(0, 1)]], axis=1) + xr[:, 0:512]
        gf = jnp.concatenate([chunk[(0, 2)], chunk[(1, 0)]], axis=1) + xr[:, 512:1024]
        gg = jnp.concatenate([chunk[(1, 1)], chunk[(1, 2)]], axis=1) + xr[:, 1024:1536]
        go = jnp.concatenate([chunk[(0, 3)], chunk[(1, 3)]], axis=1) + xr[:, 1536:2048]
        c2 = _sig(gf) * c + _sig(gi) * jnp.tanh(gg)
        h2 = _sig(go) * jnp.tanh(c2)
        return h2, c2

    def step(t, carry):
        h, c = carry
        return one_step(t, h, c)

    h, c = jax.lax.fori_loop(start + 1, end + 1, step,
                             (hc_ref[0:1, :], hc_ref[1:2, :]))
    hc_ref[0:1, :] = h
    hc_ref[1:2, :] = c
    hall_ref[...] = h.reshape(1, 1, _H)

    dummy = jnp.zeros((16, 256), jnp.bfloat16)
    for m in (0, 1):
        pltpu.matmul_acc_lhs(acc_addr=16, lhs=dummy, mxu_index=m,
                             load_staged_rhs=0)
        pltpu.matmul_acc_lhs(acc_addr=16, lhs=dummy, mxu_index=m,
                             load_staged_rhs=1)
        pltpu.matmul_pop(16, (16, 256), jnp.float32, mxu_index=m)


def _recur(starts, ends, xp3, wt):
    grid_spec = pltpu.PrefetchScalarGridSpec(
        num_scalar_prefetch=2,
        grid=(_B,),
        in_specs=[
            pl.BlockSpec((1, _T, 4 * _H), lambda b, s, e: (b, 0, 0)),
            pl.BlockSpec((16, 256, 256), lambda b, s, e: (0, 0, 0)),
        ],
        out_specs=pl.BlockSpec((1, 1, _H), lambda b, s, e: (b, 0, 0)),
        scratch_shapes=[pltpu.VMEM((2, _H), jnp.float32)],
    )
    return pl.pallas_call(
        _rec_kernel,
        grid_spec=grid_spec,
        out_shape=jax.ShapeDtypeStruct((_B, 1, _H), jnp.float32),
        compiler_params=pltpu.CompilerParams(
            dimension_semantics=("arbitrary",),
            vmem_limit_bytes=48 * 1024 * 1024,
        ),
        name="lstm_recur",
    )(starts, ends, xp3, wt)


def _mlp_kernel(h_ref, w1_ref, b1_ref, w2_ref, b2_ref, w3_ref, b3_ref,
                out_ref):
    z = jnp.maximum(
        jnp.dot(h_ref[...].astype(jnp.bfloat16), w1_ref[...],
                preferred_element_type=jnp.float32) + b1_ref[...], 0.0)
    z = jnp.maximum(
        jnp.dot(z.astype(jnp.bfloat16), w2_ref[...],
                preferred_element_type=jnp.float32) + b2_ref[...], 0.0)
    logit = jnp.sum(z * w3_ref[...], axis=1, keepdims=True) + b3_ref[...]
    out_ref[...] = jax.nn.sigmoid(logit)


def _mlp(hall, w1t, b1, w2t, b2, w3, b3):
    return pl.pallas_call(
        _mlp_kernel,
        out_shape=jax.ShapeDtypeStruct((_B, 1), jnp.float32),
        compiler_params=pltpu.CompilerParams(
            vmem_limit_bytes=48 * 1024 * 1024,
        ),
        name="lstm_mlp",
    )(hall, w1t, b1, w2t, b2, w3, b3)


def kernel(x, true_vals, W_ih, W_hh, b_ih, b_hh, W1, b1, W2, b2, W3, b3,
           starts, ends):
    bg = (b_ih + b_hh).reshape(1, 4 * _H)
    x2d = x.reshape(_B * _T, _D).astype(jnp.bfloat16)
    xp = _project(x2d, W_ih.T.astype(jnp.bfloat16), bg)
    xp3 = xp.reshape(_B, _T, 4 * _H)
    whh_t = W_hh.T.astype(jnp.bfloat16)
    wt = jnp.stack([whh_t[256 * k:256 * (k + 1), 256 * j:256 * (j + 1)]
                    for j, k in _TILE_ORDER])
    hall = _recur(starts.astype(jnp.int32), ends.astype(jnp.int32), xp3, wt)
    predict = _mlp(hall.reshape(_B, _H), W1.T.astype(jnp.bfloat16),
                   b1.reshape(1, -1), W2.T.astype(jnp.bfloat16),
                   b2.reshape(1, -1), W3.reshape(1, -1), b3.reshape(1, 1))
    return predict, true_vals
